# baseline (speedup 1.0000x reference)
import functools

import jax
import jax.numpy as jnp
from jax import lax
from jax.experimental import pallas as pl
from jax.experimental.pallas import tpu as pltpu
from jax.experimental.pallas import tpu_sc as plsc

N_CLS = 1000
PRE = 5
NCTX = 16
TOT = 77
SUF = TOT - PRE - NCTX
D = 512
NW = 32
ITERS = (N_CLS + NW - 1) // NW

PRE_W = PRE * D
CTX_W = NCTX * D
SUF_W = SUF * D
TOT_W = TOT * D

_mesh = plsc.VectorSubcoreMesh(core_axis_name="c", subcore_axis_name="s")


@functools.partial(
    pl.kernel,
    mesh=_mesh,
    out_type=jax.ShapeDtypeStruct((N_CLS, TOT_W), jnp.float32),
    scratch_types=[pltpu.VMEM((TOT_W,), jnp.float32)],
)
def _assemble(ctx_hbm, pre_hbm, suf_hbm, out_hbm, buf):
    wid = lax.axis_index("s") * 2 + lax.axis_index("c")
    pltpu.sync_copy(ctx_hbm, buf.at[pl.ds(PRE_W, CTX_W)])
    for i in range(ITERS):
        c = i * NW + wid

        @pl.when(c < N_CLS)
        def _():
            pltpu.sync_copy(pre_hbm.at[c], buf.at[pl.ds(0, PRE_W)])
            pltpu.sync_copy(suf_hbm.at[c], buf.at[pl.ds(PRE_W + CTX_W, SUF_W)])
            pltpu.sync_copy(buf, out_hbm.at[c])


def kernel(ctx, token_prefix, token_suffix):
    out = _assemble(
        ctx.reshape(CTX_W),
        token_prefix.reshape(N_CLS, PRE_W),
        token_suffix.reshape(N_CLS, SUF_W),
    )
    return out.reshape(N_CLS, TOT, D)


# speedup vs baseline: 1.0877x; 1.0877x over previous
import functools

import jax
import jax.numpy as jnp
from jax import lax
from jax.experimental import pallas as pl
from jax.experimental.pallas import tpu as pltpu
from jax.experimental.pallas import tpu_sc as plsc

N_CLS = 1000
PRE = 5
NCTX = 16
TOT = 77
SUF = TOT - PRE - NCTX
D = 512
NW = 32
ITERS = (N_CLS + NW - 1) // NW
NSLOT = 3

PRE_W = PRE * D
CTX_W = NCTX * D
SUF_W = SUF * D
TOT_W = TOT * D

_mesh = plsc.VectorSubcoreMesh(core_axis_name="c", subcore_axis_name="s")


@functools.partial(
    pl.kernel,
    mesh=_mesh,
    out_type=jax.ShapeDtypeStruct((N_CLS, TOT_W), jnp.float32),
    scratch_types=(
        [pltpu.VMEM((TOT_W,), jnp.float32)] * NSLOT
        + [pltpu.SemaphoreType.DMA] * (2 * NSLOT)
    ),
)
def _assemble(ctx_hbm, pre_hbm, suf_hbm, out_hbm,
              b0, b1, b2, si0, si1, si2, so0, so1, so2):
    bufs = [b0, b1, b2]
    sin = [si0, si1, si2]
    sout = [so0, so1, so2]
    wid = lax.axis_index("s") * 2 + lax.axis_index("c")

    for s in range(NSLOT):
        pltpu.sync_copy(ctx_hbm, bufs[s].at[pl.ds(PRE_W, CTX_W)])

    def in_copies(i):
        s = i % NSLOT
        c = i * NW + wid
        return (
            c,
            pltpu.make_async_copy(
                pre_hbm.at[c], bufs[s].at[pl.ds(0, PRE_W)], sin[s]),
            pltpu.make_async_copy(
                suf_hbm.at[c], bufs[s].at[pl.ds(PRE_W + CTX_W, SUF_W)], sin[s]),
        )

    def out_copy(i):
        s = i % NSLOT
        c = i * NW + wid
        return c, pltpu.make_async_copy(bufs[s], out_hbm.at[c], sout[s])

    def start_in(i):
        c, cp_pre, cp_suf = in_copies(i)

        @pl.when(c < N_CLS)
        def _():
            cp_pre.start()
            cp_suf.start()

    def wait_in(i):
        c, cp_pre, cp_suf = in_copies(i)

        @pl.when(c < N_CLS)
        def _():
            cp_pre.wait()
            cp_suf.wait()

    def start_out(i):
        c, cp = out_copy(i)

        @pl.when(c < N_CLS)
        def _():
            cp.start()

    def wait_out(i):
        c, cp = out_copy(i)

        @pl.when(c < N_CLS)
        def _():
            cp.wait()

    start_in(0)
    start_in(1)
    for i in range(ITERS):
        wait_in(i)
        start_out(i)
        nxt = i + 2
        if nxt < ITERS:
            if nxt >= NSLOT:
                wait_out(nxt - NSLOT)
            start_in(nxt)
    for i in range(max(0, ITERS - NSLOT), ITERS):
        wait_out(i)


def kernel(ctx, token_prefix, token_suffix):
    out = _assemble(
        ctx.reshape(CTX_W),
        token_prefix.reshape(N_CLS, PRE_W),
        token_suffix.reshape(N_CLS, SUF_W),
    )
    return out.reshape(N_CLS, TOT, D)
